# baseline (speedup 1.0000x reference)
import jax
import jax.numpy as jnp
from jax.experimental import pallas as pl
from jax.experimental.pallas import tpu as pltpu

N = 207
NP = 208
B = 32
H = 64
T = 12
BN = B * NP
DIN = 2
FW = 3 * H
F32 = jnp.float32
BF16 = jnp.bfloat16


def _dot(a, b):
    return jnp.dot(a, b, preferred_element_type=F32)


def _dcgru_body(x_ref, sup_ref,
                e0g, e0bg, e0c, e0bc,
                e1g, e1bg, e1c, e1bc,
                d0g, d0bg, d0c, d0bc,
                d1g, d1bg, d1c, d1bc,
                pw, pb,
                out_ref,
                h0, h1, dinp, rh_s, u_s, zf, tmat):
    s = sup_ref[...]
    s2 = 2.0 * _dot(s, s) - jnp.eye(NP, dtype=F32)
    tmat[...] = jnp.concatenate([s, s2], axis=0).astype(BF16)

    h0[...] = jnp.zeros_like(h0)
    h1[...] = jnp.zeros_like(h1)

    def diffuse_state(tm, src_get, off):
        for b in range(0, B, 4):
            vs = [src_get(b + j) for j in range(4)]
            dz = _dot(tm, jnp.concatenate(vs, axis=1).astype(BF16))
            for j in range(4):
                sl = slice(j * H, (j + 1) * H)
                zf[b + j, :, off:off + H] = vs[j]
                zf[b + j, :, off + H:off + 2 * H] = dz[0:NP, sl]
                zf[b + j, :, off + 2 * H:off + 3 * H] = dz[NP:, sl]

    def diffuse_x_all(tm, xall, d):
        dz = _dot(tm, xall.astype(BF16))
        for b in range(B):
            sl = slice(b * d, (b + 1) * d)
            zf[b, :, FW:FW + d] = xall[:, sl]
            zf[b, :, FW + d:FW + 2 * d] = dz[0:NP, sl]
            zf[b, :, FW + 2 * d:FW + 3 * d] = dz[NP:, sl]

    def cell(d, h_ref, fill_x, wg, bg, wc, bc):
        tm = tmat[...]
        fill_x(tm)
        diffuse_state(tm, lambda b: h_ref[b], 0)

        w = FW + 3 * d
        HB = B // 2

        def half_dot(wref, cast, lo):
            zh = zf[lo:lo + HB, :, 0:w].reshape(BN // 2, w)
            return _dot(zh.astype(BF16) if cast else zh, wref[...])

        g = jax.nn.sigmoid(
            jnp.concatenate([half_dot(wg, True, 0), half_dot(wg, True, HB)],
                            axis=0) + bg[...])
        hv = h_ref[...].reshape(BN, H)
        rh_s[...] = (g[:, 0:H] * hv).reshape(B, NP, H)
        u_s[...] = g[:, H:].reshape(B, NP, H)

        diffuse_state(tm, lambda b: rh_s[b], 0)

        c = jnp.tanh(
            jnp.concatenate([half_dot(wc, False, 0), half_dot(wc, False, HB)],
                            axis=0) + bc[...])
        u = u_s[...].reshape(BN, H)
        h_ref[...] = (c + u * (hv - c)).reshape(B, NP, H)

    def enc_step(t, carry):
        cell(DIN, h0, lambda tm: diffuse_x_all(tm, x_ref[t], DIN),
             e0g, e0bg, e0c, e0bc)
        cell(H, h1, lambda tm: diffuse_state(tm, lambda b: h0[b], FW),
             e1g, e1bg, e1c, e1bc)
        return carry
    jax.lax.fori_loop(0, T, enc_step, 0)

    dinp[...] = jnp.zeros_like(dinp)

    def dec_step(t, carry):
        cell(1, h0, lambda tm: diffuse_x_all(tm, dinp[...], 1),
             d0g, d0bg, d0c, d0bc)
        cell(H, h1, lambda tm: diffuse_state(tm, lambda b: h0[b], FW),
             d1g, d1bg, d1c, d1bc)
        proj = _dot(h1[...].reshape(BN, H), pw[...]) + pb[...]
        proj = proj.reshape(B, NP, 1)
        for b in range(B):
            pcol = proj[b]
            out_ref[t, :, b:b + 1] = pcol
            dinp[:, b:b + 1] = pcol
        return carry
    jax.lax.fori_loop(0, T, dec_step, 0)


def _pack_w(w, d, dtype):
    w3 = w.reshape(d + H, 3, -1)
    wh = w3[d:].transpose(1, 0, 2).reshape(3 * H, -1)
    wx = w3[:d].transpose(1, 0, 2).reshape(3 * d, -1)
    return jnp.concatenate([wh, wx], axis=0).astype(dtype)


def kernel(batch_x, batch_x_mark, support,
           enc0_Wg, enc0_bg, enc0_Wc, enc0_bc,
           enc1_Wg, enc1_bg, enc1_Wc, enc1_bc,
           dec0_Wg, dec0_bg, dec0_Wc, dec0_bc,
           dec1_Wg, dec1_bg, dec1_Wc, dec1_bc,
           proj_W, proj_b):
    x = batch_x.transpose(1, 2, 0, 3).reshape(T, N, B * DIN)
    x = jnp.pad(x, ((0, 0), (0, NP - N), (0, 0)))
    sup = jnp.pad(support, ((0, NP - N), (0, NP - N)))

    args = (x, sup,
            _pack_w(enc0_Wg, DIN, BF16), enc0_bg.reshape(1, -1),
            _pack_w(enc0_Wc, DIN, F32), enc0_bc.reshape(1, -1),
            _pack_w(enc1_Wg, H, BF16), enc1_bg.reshape(1, -1),
            _pack_w(enc1_Wc, H, F32), enc1_bc.reshape(1, -1),
            _pack_w(dec0_Wg, 1, BF16), dec0_bg.reshape(1, -1),
            _pack_w(dec0_Wc, 1, F32), dec0_bc.reshape(1, -1),
            _pack_w(dec1_Wg, H, BF16), dec1_bg.reshape(1, -1),
            _pack_w(dec1_Wc, H, F32), dec1_bc.reshape(1, -1),
            proj_W, proj_b.reshape(1, 1))

    out = pl.pallas_call(
        _dcgru_body,
        out_shape=jax.ShapeDtypeStruct((T, NP, B), F32),
        scratch_shapes=[
            pltpu.VMEM((B, NP, H), F32),
            pltpu.VMEM((B, NP, H), F32),
            pltpu.VMEM((NP, B), F32),
            pltpu.VMEM((B, NP, H), F32),
            pltpu.VMEM((B, NP, H), F32),
            pltpu.VMEM((B, NP, 2 * FW), F32),
            pltpu.VMEM((2 * NP, NP), BF16),
        ],
    )(*args)

    return out[:, :N, :].transpose(2, 0, 1)[..., None]


# speedup vs baseline: 1.3225x; 1.3225x over previous
import jax
import jax.numpy as jnp
from jax.experimental import pallas as pl
from jax.experimental.pallas import tpu as pltpu

N = 207
NP = 208
B = 32
H = 64
T = 12
BN = B * NP
DIN = 2
FW = 3 * H
F32 = jnp.float32
BF16 = jnp.bfloat16


def _dot(a, b):
    return jnp.dot(a, b, preferred_element_type=F32)


def _dcgru_body(x_ref, sup_ref,
                e0g, e0bg, e0c, e0bc,
                e1g, e1bg, e1c, e1bc,
                d0g, d0bg, d0c, d0bc,
                d1g, d1bg, d1c, d1bc,
                pw, pb,
                out_ref,
                h0, h1, dinp, rh_s, u_s, zf, tmat):
    s = sup_ref[...]
    s2 = 2.0 * _dot(s, s) - jnp.eye(NP, dtype=F32)
    tmat[...] = jnp.concatenate([s, s2], axis=0).astype(BF16)

    h0[...] = jnp.zeros_like(h0)
    h1[...] = jnp.zeros_like(h1)

    def diffuse_state(tm, src_get, off):
        for b in range(0, B, 2):
            va = src_get(b)
            vb = src_get(b + 1)
            dz = _dot(tm, jnp.concatenate([va, vb], axis=1).astype(BF16))
            zf[b, :, off:off + H] = va
            zf[b, :, off + H:off + 2 * H] = dz[0:NP, 0:H]
            zf[b, :, off + 2 * H:off + 3 * H] = dz[NP:, 0:H]
            zf[b + 1, :, off:off + H] = vb
            zf[b + 1, :, off + H:off + 2 * H] = dz[0:NP, H:]
            zf[b + 1, :, off + 2 * H:off + 3 * H] = dz[NP:, H:]

    def diffuse_x_all(tm, xall, d):
        dz = _dot(tm, xall.astype(BF16))
        for b in range(B):
            sl = slice(b * d, (b + 1) * d)
            zf[b, :, FW:FW + d] = xall[:, sl]
            zf[b, :, FW + d:FW + 2 * d] = dz[0:NP, sl]
            zf[b, :, FW + 2 * d:FW + 3 * d] = dz[NP:, sl]

    def cell(d, h_ref, fill_x, wg, bg, wc, bc):
        tm = tmat[...]
        fill_x(tm)
        diffuse_state(tm, lambda b: h_ref[b], 0)

        w = FW + 3 * d
        HB = B // 2

        def half_dot(wref, cast, lo):
            zh = zf[lo:lo + HB, :, 0:w].reshape(BN // 2, w)
            return _dot(zh.astype(BF16) if cast else zh, wref[...])

        g = jax.nn.sigmoid(
            jnp.concatenate([half_dot(wg, True, 0), half_dot(wg, True, HB)],
                            axis=0) + bg[...])
        hv = h_ref[...].reshape(BN, H)
        rh_s[...] = (g[:, 0:H] * hv).reshape(B, NP, H)
        u_s[...] = g[:, H:].reshape(B, NP, H)

        diffuse_state(tm, lambda b: rh_s[b], 0)

        c = jnp.tanh(
            jnp.concatenate([half_dot(wc, False, 0), half_dot(wc, False, HB)],
                            axis=0) + bc[...])
        u = u_s[...].reshape(BN, H)
        h_ref[...] = (c + u * (hv - c)).reshape(B, NP, H)

    def enc_step(t, carry):
        cell(DIN, h0, lambda tm: diffuse_x_all(tm, x_ref[t], DIN),
             e0g, e0bg, e0c, e0bc)
        cell(H, h1, lambda tm: diffuse_state(tm, lambda b: h0[b], FW),
             e1g, e1bg, e1c, e1bc)
        return carry
    jax.lax.fori_loop(0, T, enc_step, 0)

    dinp[...] = jnp.zeros_like(dinp)

    def dec_step(t, carry):
        cell(1, h0, lambda tm: diffuse_x_all(tm, dinp[...], 1),
             d0g, d0bg, d0c, d0bc)
        cell(H, h1, lambda tm: diffuse_state(tm, lambda b: h0[b], FW),
             d1g, d1bg, d1c, d1bc)
        proj = _dot(h1[...].reshape(BN, H), pw[...]) + pb[...]
        proj = proj.reshape(B, NP, 1)
        for b in range(B):
            pcol = proj[b]
            out_ref[t, :, b:b + 1] = pcol
            dinp[:, b:b + 1] = pcol
        return carry
    jax.lax.fori_loop(0, T, dec_step, 0)


def _pack_w(w, d, dtype):
    w3 = w.reshape(d + H, 3, -1)
    wh = w3[d:].transpose(1, 0, 2).reshape(3 * H, -1)
    wx = w3[:d].transpose(1, 0, 2).reshape(3 * d, -1)
    return jnp.concatenate([wh, wx], axis=0).astype(dtype)


def kernel(batch_x, batch_x_mark, support,
           enc0_Wg, enc0_bg, enc0_Wc, enc0_bc,
           enc1_Wg, enc1_bg, enc1_Wc, enc1_bc,
           dec0_Wg, dec0_bg, dec0_Wc, dec0_bc,
           dec1_Wg, dec1_bg, dec1_Wc, dec1_bc,
           proj_W, proj_b):
    x = batch_x.transpose(1, 2, 0, 3).reshape(T, N, B * DIN)
    x = jnp.pad(x, ((0, 0), (0, NP - N), (0, 0)))
    sup = jnp.pad(support, ((0, NP - N), (0, NP - N)))

    args = (x, sup,
            _pack_w(enc0_Wg, DIN, BF16), enc0_bg.reshape(1, -1),
            _pack_w(enc0_Wc, DIN, F32), enc0_bc.reshape(1, -1),
            _pack_w(enc1_Wg, H, BF16), enc1_bg.reshape(1, -1),
            _pack_w(enc1_Wc, H, F32), enc1_bc.reshape(1, -1),
            _pack_w(dec0_Wg, 1, BF16), dec0_bg.reshape(1, -1),
            _pack_w(dec0_Wc, 1, F32), dec0_bc.reshape(1, -1),
            _pack_w(dec1_Wg, H, BF16), dec1_bg.reshape(1, -1),
            _pack_w(dec1_Wc, H, F32), dec1_bc.reshape(1, -1),
            proj_W, proj_b.reshape(1, 1))

    out = pl.pallas_call(
        _dcgru_body,
        out_shape=jax.ShapeDtypeStruct((T, NP, B), F32),
        scratch_shapes=[
            pltpu.VMEM((B, NP, H), F32),
            pltpu.VMEM((B, NP, H), F32),
            pltpu.VMEM((NP, B), F32),
            pltpu.VMEM((B, NP, H), F32),
            pltpu.VMEM((B, NP, H), F32),
            pltpu.VMEM((B, NP, 2 * FW), F32),
            pltpu.VMEM((2 * NP, NP), BF16),
        ],
    )(*args)

    return out[:, :N, :].transpose(2, 0, 1)[..., None]


# speedup vs baseline: 1.3895x; 1.0507x over previous
import jax
import jax.numpy as jnp
from jax.experimental import pallas as pl
from jax.experimental.pallas import tpu as pltpu

N = 207
NP = 208
B = 32
H = 64
T = 12
BN = B * NP
DIN = 2
FW = 3 * H
L1B = 256
F32 = jnp.float32
BF16 = jnp.bfloat16


def _dot(a, b):
    return jnp.dot(a, b, preferred_element_type=F32)


def _dcgru_body(x_ref, sup_ref,
                e0g, e0bg, e0c, e0bc,
                e1g, e1bg, e1c, e1bc,
                d0g, d0bg, d0c, d0bc,
                d1g, d1bg, d1c, d1bc,
                pw, pb,
                out_ref,
                h0, h1, dinp, rh_s, u_s, zf, tmat):
    s = sup_ref[...]
    s2 = 2.0 * _dot(s, s) - jnp.eye(NP, dtype=F32)
    tmat[...] = jnp.concatenate([s, s2], axis=0).astype(BF16)

    h0[...] = jnp.zeros_like(h0)
    h1[...] = jnp.zeros_like(h1)

    def diffuse_state(tm, src_get, off):
        for b in range(0, B, 2):
            va = src_get(b)
            vb = src_get(b + 1)
            dz = _dot(tm, jnp.concatenate([va, vb], axis=1).astype(BF16))
            zf[b, :, off:off + H] = va
            zf[b, :, off + H:off + 2 * H] = dz[0:NP, 0:H]
            zf[b, :, off + 2 * H:off + 3 * H] = dz[NP:, 0:H]
            zf[b + 1, :, off:off + H] = vb
            zf[b + 1, :, off + H:off + 2 * H] = dz[0:NP, H:]
            zf[b + 1, :, off + 2 * H:off + 3 * H] = dz[NP:, H:]

    def diffuse_x_all(tm, xall, d):
        dz = _dot(tm, xall.astype(BF16))
        for b in range(B):
            sl = slice(b * d, (b + 1) * d)
            zf[b, :, FW:FW + d] = xall[:, sl]
            zf[b, :, FW + d:FW + 2 * d] = dz[0:NP, sl]
            zf[b, :, FW + 2 * d:FW + 3 * d] = dz[NP:, sl]

    def cell(d, base, h_ref, wg, bg, wc, bc):
        tm = tmat[...]
        w = FW + 3 * d
        zv = zf[:, :, base:base + w].reshape(BN, w)
        g = jax.nn.sigmoid(_dot(zv.astype(BF16), wg[...]) + bg[...])
        hv = h_ref[...].reshape(BN, H)
        rh_s[...] = (g[:, 0:H] * hv).reshape(B, NP, H)
        u_s[...] = g[:, H:].reshape(B, NP, H)

        diffuse_state(tm, lambda b: rh_s[b], base)

        zv2 = zf[:, :, base:base + w].reshape(BN, w)
        c = jnp.tanh(_dot(zv2, wc[...]) + bc[...])
        u = u_s[...].reshape(BN, H)
        h_ref[...] = (c + u * (hv - c)).reshape(B, NP, H)

    def step(t, x_fill, g0, bg0, c0, bc0, g1, bg1, c1, bc1, d):
        tm = tmat[...]
        diffuse_state(tm, lambda b: h1[b], L1B)
        x_fill(tm)
        cell(d, 0, h0, g0, bg0, c0, bc0)
        diffuse_state(tm, lambda b: h0[b], L1B + FW)
        diffuse_state(tm, lambda b: h0[b], 0)
        cell(H, L1B, h1, g1, bg1, c1, bc1)

    zf[...] = jnp.zeros_like(zf)

    def enc_step(t, carry):
        step(t, lambda tm: diffuse_x_all(tm, x_ref[t], DIN),
             e0g, e0bg, e0c, e0bc, e1g, e1bg, e1c, e1bc, DIN)
        return carry
    jax.lax.fori_loop(0, T, enc_step, 0)

    dinp[...] = jnp.zeros_like(dinp)

    def dec_step(t, carry):
        step(t, lambda tm: diffuse_x_all(tm, dinp[...], 1),
             d0g, d0bg, d0c, d0bc, d1g, d1bg, d1c, d1bc, 1)
        proj = _dot(h1[...].reshape(BN, H), pw[...]) + pb[...]
        proj = proj.reshape(B, NP, 1)
        for b in range(B):
            pcol = proj[b]
            out_ref[t, :, b:b + 1] = pcol
            dinp[:, b:b + 1] = pcol
        return carry
    jax.lax.fori_loop(0, T, dec_step, 0)


def _pack_w(w, d, dtype):
    w3 = w.reshape(d + H, 3, -1)
    wh = w3[d:].transpose(1, 0, 2).reshape(3 * H, -1)
    wx = w3[:d].transpose(1, 0, 2).reshape(3 * d, -1)
    return jnp.concatenate([wh, wx], axis=0).astype(dtype)


def kernel(batch_x, batch_x_mark, support,
           enc0_Wg, enc0_bg, enc0_Wc, enc0_bc,
           enc1_Wg, enc1_bg, enc1_Wc, enc1_bc,
           dec0_Wg, dec0_bg, dec0_Wc, dec0_bc,
           dec1_Wg, dec1_bg, dec1_Wc, dec1_bc,
           proj_W, proj_b):
    x = batch_x.transpose(1, 2, 0, 3).reshape(T, N, B * DIN)
    x = jnp.pad(x, ((0, 0), (0, NP - N), (0, 0)))
    sup = jnp.pad(support, ((0, NP - N), (0, NP - N)))

    args = (x, sup,
            _pack_w(enc0_Wg, DIN, BF16), enc0_bg.reshape(1, -1),
            _pack_w(enc0_Wc, DIN, F32), enc0_bc.reshape(1, -1),
            _pack_w(enc1_Wg, H, BF16), enc1_bg.reshape(1, -1),
            _pack_w(enc1_Wc, H, F32), enc1_bc.reshape(1, -1),
            _pack_w(dec0_Wg, 1, BF16), dec0_bg.reshape(1, -1),
            _pack_w(dec0_Wc, 1, F32), dec0_bc.reshape(1, -1),
            _pack_w(dec1_Wg, H, BF16), dec1_bg.reshape(1, -1),
            _pack_w(dec1_Wc, H, F32), dec1_bc.reshape(1, -1),
            proj_W, proj_b.reshape(1, 1))

    out = pl.pallas_call(
        _dcgru_body,
        out_shape=jax.ShapeDtypeStruct((T, NP, B), F32),
        scratch_shapes=[
            pltpu.VMEM((B, NP, H), F32),
            pltpu.VMEM((B, NP, H), F32),
            pltpu.VMEM((NP, B), F32),
            pltpu.VMEM((B, NP, H), F32),
            pltpu.VMEM((B, NP, H), F32),
            pltpu.VMEM((B, NP, L1B + 2 * FW), F32),
            pltpu.VMEM((2 * NP, NP), BF16),
        ],
    )(*args)

    return out[:, :N, :].transpose(2, 0, 1)[..., None]
